# speedup vs baseline: 4.0691x; 4.0691x over previous
import functools
import math

import jax
import jax.numpy as jnp
from jax import lax
from jax.experimental import pallas as pl
from jax.experimental.pallas import tpu as pltpu
from jax.experimental.pallas import tpu_sc as plsc

NUM_ROADS = 512
N_LANE = 50000
N_ROAD = 50000
D = 128
N_PATH = 25000


_A_BLK = 2000
_A_GRID = N_ROAD // _A_BLK


def _seg_mean_body(feat_ref, idx_ref, out_ref, acc_sum, acc_cnt):
    i = pl.program_id(0)

    @pl.when(i == 0)
    def _init():
        acc_sum[...] = jnp.zeros_like(acc_sum)
        acc_cnt[...] = jnp.zeros_like(acc_cnt)

    idx = idx_ref[0]
    iota = lax.broadcasted_iota(jnp.int32, (NUM_ROADS, _A_BLK), 0)
    onehot_t = (iota == idx).astype(jnp.float32)
    acc_sum[...] += lax.dot_general(
        onehot_t, feat_ref[...], (((1,), (0,)), ((), ())),
        preferred_element_type=jnp.float32)
    cnt = jnp.sum(onehot_t, axis=1, keepdims=True)
    acc_cnt[...] += jnp.broadcast_to(cnt, acc_cnt.shape)

    @pl.when(i == _A_GRID - 1)
    def _fin():
        out_ref[...] = acc_sum[...] / jnp.clip(acc_cnt[...], 1.0, None)


def _road_mean_tc(road_feat, road_idx):
    idx3 = road_idx.reshape(_A_GRID, 1, _A_BLK)
    return pl.pallas_call(
        _seg_mean_body,
        grid=(_A_GRID,),
        in_specs=[
            pl.BlockSpec((_A_BLK, D), lambda i: (i, 0)),
            pl.BlockSpec((1, 1, _A_BLK), lambda i: (i, 0, 0)),
        ],
        out_specs=pl.BlockSpec((NUM_ROADS, D), lambda i: (0, 0)),
        out_shape=jax.ShapeDtypeStruct((NUM_ROADS, D), jnp.float32),
        scratch_shapes=[
            pltpu.VMEM((NUM_ROADS, D), jnp.float32),
            pltpu.VMEM((NUM_ROADS, D), jnp.float32),
        ],
    )(road_feat, idx3)


_B_BLK = 2000
_B_GRID = N_LANE // _B_BLK
_SCALE = 1.0 / math.sqrt(D)


def _sim_body(lane_ref, mean_ref, sim_ref, soft_ref):
    sim = lax.dot_general(
        lane_ref[...], mean_ref[...], (((1,), (1,)), ((), ())),
        preferred_element_type=jnp.float32) * _SCALE
    sim_ref[...] = sim
    m = jnp.max(sim, axis=1, keepdims=True)
    e = jnp.exp(sim - m)
    s = jnp.sum(e, axis=1, keepdims=True)
    soft_ref[...] = e / s


def _sim_tc(lane_feat, road_mean):
    return pl.pallas_call(
        _sim_body,
        grid=(_B_GRID,),
        in_specs=[
            pl.BlockSpec((_B_BLK, D), lambda i: (i, 0)),
            pl.BlockSpec((NUM_ROADS, D), lambda i: (0, 0)),
        ],
        out_specs=[
            pl.BlockSpec((_B_BLK, NUM_ROADS), lambda i: (i, 0)),
            pl.BlockSpec((_B_BLK, NUM_ROADS), lambda i: (i, 0)),
        ],
        out_shape=[
            jax.ShapeDtypeStruct((N_LANE, NUM_ROADS), jnp.float32),
            jax.ShapeDtypeStruct((N_LANE, NUM_ROADS), jnp.float32),
        ],
    )(lane_feat, road_mean)


_NC = 2
_NS = 16
_NW = _NC * _NS
_BPW = 784
_GCHUNK = 112
_NCHUNK = _BPW // _GCHUNK


def _path_gather_sc(sim, path_inverse, gt):
    mesh = plsc.VectorSubcoreMesh(core_axis_name="c", subcore_axis_name="s")

    @functools.partial(
        pl.kernel,
        mesh=mesh,
        out_type=[
            jax.ShapeDtypeStruct((N_PATH, NUM_ROADS), jnp.float32),
            jax.ShapeDtypeStruct((N_PATH,), jnp.int32),
        ],
        scratch_types=[
            pltpu.VMEM((_BPW,), jnp.int32),
            pltpu.VMEM((_GCHUNK, NUM_ROADS), jnp.float32),
            pltpu.VMEM((_BPW,), jnp.int32),
            pltpu.SemaphoreType.DMA,
        ],
    )
    def k(sim_hbm, pinv_hbm, gt_hbm, psim_hbm, pgt_hbm, idx_v, rows_v, gt_v,
          sem):
        wid = lax.axis_index("s") * _NC + lax.axis_index("c")
        base = jnp.minimum(wid * _BPW, N_PATH - _BPW)
        pltpu.sync_copy(pinv_hbm.at[pl.ds(base, _BPW)], idx_v)
        pltpu.async_copy(gt_hbm.at[idx_v], gt_v, sem).wait()
        pltpu.sync_copy(gt_v, pgt_hbm.at[pl.ds(base, _BPW)])
        for c in range(_NCHUNK):
            pltpu.async_copy(
                sim_hbm.at[idx_v.at[pl.ds(c * _GCHUNK, _GCHUNK)]],
                rows_v, sem).wait()
            pltpu.sync_copy(
                rows_v, psim_hbm.at[pl.ds(base + c * _GCHUNK, _GCHUNK)])

    return k(sim, path_inverse, gt)


def kernel(lane_feat, road_feat, road_idx, path_inverse, gt):
    road_mean = _road_mean_tc(road_feat, road_idx)
    sim, sim_softmax = _sim_tc(lane_feat, road_mean)
    path_sim, path_gt = _path_gather_sc(sim, path_inverse, gt)
    return sim, sim_softmax, path_sim, path_gt


# speedup vs baseline: 4.3297x; 1.0640x over previous
import functools
import math

import jax
import jax.numpy as jnp
from jax import lax
from jax.experimental import pallas as pl
from jax.experimental.pallas import tpu as pltpu
from jax.experimental.pallas import tpu_sc as plsc

NUM_ROADS = 512
N_LANE = 50000
N_ROAD = 50000
D = 128
N_PATH = 25000


_A_BLK = 2000
_A_GRID = N_ROAD // _A_BLK


def _seg_mean_body(feat_ref, idx_ref, out_ref, acc_sum, acc_cnt):
    i = pl.program_id(0)

    @pl.when(i == 0)
    def _init():
        acc_sum[...] = jnp.zeros_like(acc_sum)
        acc_cnt[...] = jnp.zeros_like(acc_cnt)

    idx = idx_ref[0]
    iota = lax.broadcasted_iota(jnp.int32, (NUM_ROADS, _A_BLK), 0)
    onehot_t = (iota == idx).astype(jnp.float32)
    acc_sum[...] += lax.dot_general(
        onehot_t, feat_ref[...], (((1,), (0,)), ((), ())),
        preferred_element_type=jnp.float32)
    cnt = jnp.sum(onehot_t, axis=1, keepdims=True)
    acc_cnt[...] += jnp.broadcast_to(cnt, acc_cnt.shape)

    @pl.when(i == _A_GRID - 1)
    def _fin():
        out_ref[...] = acc_sum[...] / jnp.clip(acc_cnt[...], 1.0, None)


def _road_mean_tc(road_feat, road_idx):
    idx3 = road_idx.reshape(_A_GRID, 1, _A_BLK)
    return pl.pallas_call(
        _seg_mean_body,
        grid=(_A_GRID,),
        in_specs=[
            pl.BlockSpec((_A_BLK, D), lambda i: (i, 0)),
            pl.BlockSpec((1, 1, _A_BLK), lambda i: (i, 0, 0)),
        ],
        out_specs=pl.BlockSpec((NUM_ROADS, D), lambda i: (0, 0)),
        out_shape=jax.ShapeDtypeStruct((NUM_ROADS, D), jnp.float32),
        scratch_shapes=[
            pltpu.VMEM((NUM_ROADS, D), jnp.float32),
            pltpu.VMEM((NUM_ROADS, D), jnp.float32),
        ],
    )(road_feat, idx3)


_B_BLK = 2000
_B_GRID = N_LANE // _B_BLK
_SCALE = 1.0 / math.sqrt(D)


def _sim_body(lane_ref, mean_ref, sim_ref, soft_ref):
    sim = lax.dot_general(
        lane_ref[...], mean_ref[...], (((1,), (1,)), ((), ())),
        preferred_element_type=jnp.float32) * _SCALE
    sim_ref[...] = sim
    m = jnp.max(sim, axis=1, keepdims=True)
    e = jnp.exp(sim - m)
    s = jnp.sum(e, axis=1, keepdims=True)
    soft_ref[...] = e / s


def _sim_tc(lane_feat, road_mean):
    return pl.pallas_call(
        _sim_body,
        grid=(_B_GRID,),
        in_specs=[
            pl.BlockSpec((_B_BLK, D), lambda i: (i, 0)),
            pl.BlockSpec((NUM_ROADS, D), lambda i: (0, 0)),
        ],
        out_specs=[
            pl.BlockSpec((_B_BLK, NUM_ROADS), lambda i: (i, 0)),
            pl.BlockSpec((_B_BLK, NUM_ROADS), lambda i: (i, 0)),
        ],
        out_shape=[
            jax.ShapeDtypeStruct((N_LANE, NUM_ROADS), jnp.float32),
            jax.ShapeDtypeStruct((N_LANE, NUM_ROADS), jnp.float32),
        ],
    )(lane_feat, road_mean)


_NC = 2
_NS = 16
_NW = _NC * _NS
_BPW = 784


def _path_gather_sc(lane_feat, path_inverse, gt):
    mesh = plsc.VectorSubcoreMesh(core_axis_name="c", subcore_axis_name="s")

    @functools.partial(
        pl.kernel,
        mesh=mesh,
        out_type=[
            jax.ShapeDtypeStruct((N_PATH, D), jnp.float32),
            jax.ShapeDtypeStruct((N_PATH,), jnp.int32),
        ],
        scratch_types=[
            pltpu.VMEM((_BPW,), jnp.int32),
            pltpu.VMEM((_BPW, D), jnp.float32),
            pltpu.VMEM((_BPW,), jnp.int32),
            pltpu.SemaphoreType.DMA,
        ],
    )
    def k(lane_hbm, pinv_hbm, gt_hbm, plane_hbm, pgt_hbm, idx_v, rows_v,
          gt_v, sem):
        wid = lax.axis_index("s") * _NC + lax.axis_index("c")
        base = jnp.minimum(wid * _BPW, N_PATH - _BPW)
        pltpu.sync_copy(pinv_hbm.at[pl.ds(base, _BPW)], idx_v)
        gt_cp = pltpu.async_copy(gt_hbm.at[idx_v], gt_v, sem)
        rows_cp = pltpu.async_copy(lane_hbm.at[idx_v], rows_v, sem)
        gt_cp.wait()
        rows_cp.wait()
        pltpu.sync_copy(gt_v, pgt_hbm.at[pl.ds(base, _BPW)])
        pltpu.sync_copy(rows_v, plane_hbm.at[pl.ds(base, _BPW)])

    return k(lane_feat, path_inverse, gt)


_P_BLK = 1000
_P_GRID = N_PATH // _P_BLK


def _psim_body(plane_ref, mean_ref, out_ref):
    out_ref[...] = lax.dot_general(
        plane_ref[...], mean_ref[...], (((1,), (1,)), ((), ())),
        preferred_element_type=jnp.float32) * _SCALE


def _psim_tc(path_lane, road_mean):
    return pl.pallas_call(
        _psim_body,
        grid=(_P_GRID,),
        in_specs=[
            pl.BlockSpec((_P_BLK, D), lambda i: (i, 0)),
            pl.BlockSpec((NUM_ROADS, D), lambda i: (0, 0)),
        ],
        out_specs=pl.BlockSpec((_P_BLK, NUM_ROADS), lambda i: (i, 0)),
        out_shape=jax.ShapeDtypeStruct((N_PATH, NUM_ROADS), jnp.float32),
    )(path_lane, road_mean)


def kernel(lane_feat, road_feat, road_idx, path_inverse, gt):
    path_lane, path_gt = _path_gather_sc(lane_feat, path_inverse, gt)
    road_mean = _road_mean_tc(road_feat, road_idx)
    sim, sim_softmax = _sim_tc(lane_feat, road_mean)
    path_sim = _psim_tc(path_lane, road_mean)
    return sim, sim_softmax, path_sim, path_gt


# speedup vs baseline: 4.6426x; 1.0723x over previous
import functools
import math

import jax
import jax.numpy as jnp
from jax import lax
from jax.experimental import pallas as pl
from jax.experimental.pallas import tpu as pltpu
from jax.experimental.pallas import tpu_sc as plsc

NUM_ROADS = 512
N_LANE = 50000
N_ROAD = 50000
D = 128
N_PATH = 25000


_A_BLK = 2000
_A_GRID = N_ROAD // _A_BLK


def _seg_mean_body(feat_ref, idx_ref, out_ref, acc_sum, acc_cnt):
    i = pl.program_id(0)

    @pl.when(i == 0)
    def _init():
        acc_sum[...] = jnp.zeros_like(acc_sum)
        acc_cnt[...] = jnp.zeros_like(acc_cnt)

    idx = idx_ref[0]
    iota = lax.broadcasted_iota(jnp.int32, (NUM_ROADS, _A_BLK), 0)
    onehot_t = (iota == idx).astype(jnp.float32)
    acc_sum[...] += lax.dot_general(
        onehot_t, feat_ref[...], (((1,), (0,)), ((), ())),
        preferred_element_type=jnp.float32)
    cnt = jnp.sum(onehot_t, axis=1, keepdims=True)
    acc_cnt[...] += jnp.broadcast_to(cnt, acc_cnt.shape)

    @pl.when(i == _A_GRID - 1)
    def _fin():
        out_ref[...] = acc_sum[...] / jnp.clip(acc_cnt[...], 1.0, None)


def _road_mean_tc(road_feat, road_idx):
    idx3 = road_idx.reshape(_A_GRID, 1, _A_BLK)
    return pl.pallas_call(
        _seg_mean_body,
        grid=(_A_GRID,),
        in_specs=[
            pl.BlockSpec((_A_BLK, D), lambda i: (i, 0)),
            pl.BlockSpec((1, 1, _A_BLK), lambda i: (i, 0, 0)),
        ],
        out_specs=pl.BlockSpec((NUM_ROADS, D), lambda i: (0, 0)),
        out_shape=jax.ShapeDtypeStruct((NUM_ROADS, D), jnp.float32),
        scratch_shapes=[
            pltpu.VMEM((NUM_ROADS, D), jnp.float32),
            pltpu.VMEM((NUM_ROADS, D), jnp.float32),
        ],
    )(road_feat, idx3)


_B_BLK = 5000
_B_GRID = N_LANE // _B_BLK
_SCALE = 1.0 / math.sqrt(D)


def _sim_body(lane_ref, mean_ref, sim_ref, soft_ref):
    sim = lax.dot_general(
        lane_ref[...], mean_ref[...], (((1,), (1,)), ((), ())),
        preferred_element_type=jnp.float32) * _SCALE
    sim_ref[...] = sim
    m = jnp.max(sim, axis=1, keepdims=True)
    e = jnp.exp(sim - m)
    s = jnp.sum(e, axis=1, keepdims=True)
    soft_ref[...] = e / s


def _sim_tc(lane_feat, road_mean):
    return pl.pallas_call(
        _sim_body,
        grid=(_B_GRID,),
        in_specs=[
            pl.BlockSpec((_B_BLK, D), lambda i: (i, 0)),
            pl.BlockSpec((NUM_ROADS, D), lambda i: (0, 0)),
        ],
        out_specs=[
            pl.BlockSpec((_B_BLK, NUM_ROADS), lambda i: (i, 0)),
            pl.BlockSpec((_B_BLK, NUM_ROADS), lambda i: (i, 0)),
        ],
        out_shape=[
            jax.ShapeDtypeStruct((N_LANE, NUM_ROADS), jnp.float32),
            jax.ShapeDtypeStruct((N_LANE, NUM_ROADS), jnp.float32),
        ],
    )(lane_feat, road_mean)


_NC = 2
_NS = 16
_NW = _NC * _NS
_BPW = 784


def _path_gather_sc(lane_feat, path_inverse, gt):
    mesh = plsc.VectorSubcoreMesh(core_axis_name="c", subcore_axis_name="s")

    @functools.partial(
        pl.kernel,
        mesh=mesh,
        out_type=[
            jax.ShapeDtypeStruct((N_PATH, D), jnp.float32),
            jax.ShapeDtypeStruct((N_PATH,), jnp.int32),
        ],
        scratch_types=[
            pltpu.VMEM((_BPW,), jnp.int32),
            pltpu.VMEM((_BPW, D), jnp.float32),
            pltpu.VMEM((_BPW,), jnp.int32),
            pltpu.SemaphoreType.DMA,
        ],
    )
    def k(lane_hbm, pinv_hbm, gt_hbm, plane_hbm, pgt_hbm, idx_v, rows_v,
          gt_v, sem):
        wid = lax.axis_index("s") * _NC + lax.axis_index("c")
        base = jnp.minimum(wid * _BPW, N_PATH - _BPW)
        pltpu.sync_copy(pinv_hbm.at[pl.ds(base, _BPW)], idx_v)
        gt_cp = pltpu.async_copy(gt_hbm.at[idx_v], gt_v, sem)
        rows_cp = pltpu.async_copy(lane_hbm.at[idx_v], rows_v, sem)
        gt_cp.wait()
        rows_cp.wait()
        pltpu.sync_copy(gt_v, pgt_hbm.at[pl.ds(base, _BPW)])
        pltpu.sync_copy(rows_v, plane_hbm.at[pl.ds(base, _BPW)])

    return k(lane_feat, path_inverse, gt)


_P_BLK = 5000
_P_GRID = N_PATH // _P_BLK


def _psim_body(plane_ref, mean_ref, out_ref):
    out_ref[...] = lax.dot_general(
        plane_ref[...], mean_ref[...], (((1,), (1,)), ((), ())),
        preferred_element_type=jnp.float32) * _SCALE


def _psim_tc(path_lane, road_mean):
    return pl.pallas_call(
        _psim_body,
        grid=(_P_GRID,),
        in_specs=[
            pl.BlockSpec((_P_BLK, D), lambda i: (i, 0)),
            pl.BlockSpec((NUM_ROADS, D), lambda i: (0, 0)),
        ],
        out_specs=pl.BlockSpec((_P_BLK, NUM_ROADS), lambda i: (i, 0)),
        out_shape=jax.ShapeDtypeStruct((N_PATH, NUM_ROADS), jnp.float32),
    )(path_lane, road_mean)


def kernel(lane_feat, road_feat, road_idx, path_inverse, gt):
    path_lane, path_gt = _path_gather_sc(lane_feat, path_inverse, gt)
    road_mean = _road_mean_tc(road_feat, road_idx)
    sim, sim_softmax = _sim_tc(lane_feat, road_mean)
    path_sim = _psim_tc(path_lane, road_mean)
    return sim, sim_softmax, path_sim, path_gt


# speedup vs baseline: 4.9384x; 1.0637x over previous
import functools
import math

import jax
import jax.numpy as jnp
from jax import lax
from jax.experimental import pallas as pl
from jax.experimental.pallas import tpu as pltpu
from jax.experimental.pallas import tpu_sc as plsc

NUM_ROADS = 512
N_LANE = 50000
N_ROAD = 50000
D = 128
N_PATH = 25000


_A_BLK = 5000
_A_GRID = N_ROAD // _A_BLK
_B_BLK = 5000
_B_GRID = N_LANE // _B_BLK
_SCALE = 1.0 / math.sqrt(D)


def _ab_body(road_ref, idx_ref, lane_ref, mean_ref, sim_ref, soft_ref,
             acc_sum, acc_cnt, mean_s):
    i = pl.program_id(0)

    @pl.when(i == 0)
    def _init():
        acc_sum[...] = jnp.zeros_like(acc_sum)
        acc_cnt[...] = jnp.zeros_like(acc_cnt)

    @pl.when(i < _A_GRID)
    def _seg():
        idx = idx_ref[0]
        iota = lax.broadcasted_iota(jnp.int32, (NUM_ROADS, _A_BLK), 0)
        onehot_t = (iota == idx).astype(jnp.float32)
        acc_sum[...] += lax.dot_general(
            onehot_t, road_ref[...], (((1,), (0,)), ((), ())),
            preferred_element_type=jnp.float32)
        cnt = jnp.sum(onehot_t, axis=1, keepdims=True)
        acc_cnt[...] += jnp.broadcast_to(cnt, acc_cnt.shape)

    @pl.when(i == _A_GRID)
    def _mean():
        m = acc_sum[...] / jnp.clip(acc_cnt[...], 1.0, None)
        mean_s[...] = m
        mean_ref[...] = m

    @pl.when(i >= _A_GRID)
    def _sim():
        sim = lax.dot_general(
            lane_ref[...], mean_s[...], (((1,), (1,)), ((), ())),
            preferred_element_type=jnp.float32) * _SCALE
        sim_ref[...] = sim
        mx = jnp.max(sim, axis=1, keepdims=True)
        e = jnp.exp(sim - mx)
        s = jnp.sum(e, axis=1, keepdims=True)
        soft_ref[...] = e / s


def _seg_sim_tc(road_feat, road_idx, lane_feat):
    idx3 = road_idx.reshape(_A_GRID, 1, _A_BLK)
    grid = _A_GRID + _B_GRID
    return pl.pallas_call(
        _ab_body,
        grid=(grid,),
        in_specs=[
            pl.BlockSpec((_A_BLK, D), lambda i: (jnp.minimum(i, _A_GRID - 1), 0)),
            pl.BlockSpec((1, 1, _A_BLK),
                         lambda i: (jnp.minimum(i, _A_GRID - 1), 0, 0)),
            pl.BlockSpec((_B_BLK, D),
                         lambda i: (jnp.maximum(i - _A_GRID, 0), 0)),
        ],
        out_specs=[
            pl.BlockSpec((NUM_ROADS, D), lambda i: (0, 0)),
            pl.BlockSpec((_B_BLK, NUM_ROADS),
                         lambda i: (jnp.maximum(i - _A_GRID, 0), 0)),
            pl.BlockSpec((_B_BLK, NUM_ROADS),
                         lambda i: (jnp.maximum(i - _A_GRID, 0), 0)),
        ],
        out_shape=[
            jax.ShapeDtypeStruct((NUM_ROADS, D), jnp.float32),
            jax.ShapeDtypeStruct((N_LANE, NUM_ROADS), jnp.float32),
            jax.ShapeDtypeStruct((N_LANE, NUM_ROADS), jnp.float32),
        ],
        scratch_shapes=[
            pltpu.VMEM((NUM_ROADS, D), jnp.float32),
            pltpu.VMEM((NUM_ROADS, D), jnp.float32),
            pltpu.VMEM((NUM_ROADS, D), jnp.float32),
        ],
    )(road_feat, idx3, lane_feat)


_NC = 2
_NS = 16
_NW = _NC * _NS
_BPW = 784


def _path_gather_sc(lane_feat, path_inverse, gt):
    mesh = plsc.VectorSubcoreMesh(core_axis_name="c", subcore_axis_name="s")

    @functools.partial(
        pl.kernel,
        mesh=mesh,
        out_type=[
            jax.ShapeDtypeStruct((N_PATH, D), jnp.float32),
            jax.ShapeDtypeStruct((N_PATH,), jnp.int32),
        ],
        scratch_types=[
            pltpu.VMEM((_BPW,), jnp.int32),
            pltpu.VMEM((_BPW, D), jnp.float32),
            pltpu.VMEM((_BPW,), jnp.int32),
            pltpu.SemaphoreType.DMA,
        ],
    )
    def k(lane_hbm, pinv_hbm, gt_hbm, plane_hbm, pgt_hbm, idx_v, rows_v,
          gt_v, sem):
        wid = lax.axis_index("s") * _NC + lax.axis_index("c")
        base = jnp.minimum(wid * _BPW, N_PATH - _BPW)
        pltpu.sync_copy(pinv_hbm.at[pl.ds(base, _BPW)], idx_v)
        gt_cp = pltpu.async_copy(gt_hbm.at[idx_v], gt_v, sem)
        rows_cp = pltpu.async_copy(lane_hbm.at[idx_v], rows_v, sem)
        gt_cp.wait()
        rows_cp.wait()
        pltpu.sync_copy(gt_v, pgt_hbm.at[pl.ds(base, _BPW)])
        pltpu.sync_copy(rows_v, plane_hbm.at[pl.ds(base, _BPW)])

    return k(lane_feat, path_inverse, gt)


_P_BLK = 5000
_P_GRID = N_PATH // _P_BLK


def _psim_body(plane_ref, mean_ref, out_ref):
    out_ref[...] = lax.dot_general(
        plane_ref[...], mean_ref[...], (((1,), (1,)), ((), ())),
        preferred_element_type=jnp.float32) * _SCALE


def _psim_tc(path_lane, road_mean):
    return pl.pallas_call(
        _psim_body,
        grid=(_P_GRID,),
        in_specs=[
            pl.BlockSpec((_P_BLK, D), lambda i: (i, 0)),
            pl.BlockSpec((NUM_ROADS, D), lambda i: (0, 0)),
        ],
        out_specs=pl.BlockSpec((_P_BLK, NUM_ROADS), lambda i: (i, 0)),
        out_shape=jax.ShapeDtypeStruct((N_PATH, NUM_ROADS), jnp.float32),
    )(path_lane, road_mean)


def kernel(lane_feat, road_feat, road_idx, path_inverse, gt):
    path_lane, path_gt = _path_gather_sc(lane_feat, path_inverse, gt)
    road_mean, sim, sim_softmax = _seg_sim_tc(road_feat, road_idx, lane_feat)
    path_sim = _psim_tc(path_lane, road_mean)
    return sim, sim_softmax, path_sim, path_gt


# speedup vs baseline: 4.9602x; 1.0044x over previous
import functools
import math

import jax
import jax.numpy as jnp
from jax import lax
from jax.experimental import pallas as pl
from jax.experimental.pallas import tpu as pltpu
from jax.experimental.pallas import tpu_sc as plsc

NUM_ROADS = 512
N_LANE = 50000
N_ROAD = 50000
D = 128
N_PATH = 25000


_A_BLK = 5000
_A_GRID = N_ROAD // _A_BLK
_B_BLK = 5000
_B_GRID = N_LANE // _B_BLK
_SCALE = 1.0 / math.sqrt(D)


def _ab_body(road_ref, idx_ref, lane_ref, mean_ref, sim_ref, soft_ref,
             acc_sum, acc_cnt, mean_s):
    i = pl.program_id(0)

    @pl.when(i == 0)
    def _init():
        acc_sum[...] = jnp.zeros_like(acc_sum)
        acc_cnt[...] = jnp.zeros_like(acc_cnt)

    @pl.when(i < _A_GRID)
    def _seg():
        idx = idx_ref[0]
        iota = lax.broadcasted_iota(jnp.int32, (NUM_ROADS, _A_BLK), 0)
        mask = iota == idx
        onehot_t = mask.astype(jnp.bfloat16)
        acc_sum[...] += lax.dot_general(
            onehot_t, road_ref[...].astype(jnp.bfloat16),
            (((1,), (0,)), ((), ())), preferred_element_type=jnp.float32)
        cnt = jnp.sum(mask.astype(jnp.float32), axis=1, keepdims=True)
        acc_cnt[...] += jnp.broadcast_to(cnt, acc_cnt.shape)

    @pl.when(i == _A_GRID)
    def _mean():
        m = acc_sum[...] / jnp.clip(acc_cnt[...], 1.0, None)
        mean_s[...] = m
        mean_ref[...] = m

    @pl.when(i >= _A_GRID)
    def _sim():
        sim = lax.dot_general(
            lane_ref[...], mean_s[...], (((1,), (1,)), ((), ())),
            preferred_element_type=jnp.float32) * _SCALE
        sim_ref[...] = sim
        mx = jnp.max(sim, axis=1, keepdims=True)
        e = jnp.exp(sim - mx)
        s = jnp.sum(e, axis=1, keepdims=True)
        soft_ref[...] = e / s


def _seg_sim_tc(road_feat, road_idx, lane_feat):
    idx3 = road_idx.reshape(_A_GRID, 1, _A_BLK)
    grid = _A_GRID + _B_GRID
    return pl.pallas_call(
        _ab_body,
        grid=(grid,),
        in_specs=[
            pl.BlockSpec((_A_BLK, D), lambda i: (jnp.minimum(i, _A_GRID - 1), 0)),
            pl.BlockSpec((1, 1, _A_BLK),
                         lambda i: (jnp.minimum(i, _A_GRID - 1), 0, 0)),
            pl.BlockSpec((_B_BLK, D),
                         lambda i: (jnp.maximum(i - _A_GRID, 0), 0)),
        ],
        out_specs=[
            pl.BlockSpec((NUM_ROADS, D), lambda i: (0, 0)),
            pl.BlockSpec((_B_BLK, NUM_ROADS),
                         lambda i: (jnp.maximum(i - _A_GRID, 0), 0)),
            pl.BlockSpec((_B_BLK, NUM_ROADS),
                         lambda i: (jnp.maximum(i - _A_GRID, 0), 0)),
        ],
        out_shape=[
            jax.ShapeDtypeStruct((NUM_ROADS, D), jnp.float32),
            jax.ShapeDtypeStruct((N_LANE, NUM_ROADS), jnp.float32),
            jax.ShapeDtypeStruct((N_LANE, NUM_ROADS), jnp.float32),
        ],
        scratch_shapes=[
            pltpu.VMEM((NUM_ROADS, D), jnp.float32),
            pltpu.VMEM((NUM_ROADS, D), jnp.float32),
            pltpu.VMEM((NUM_ROADS, D), jnp.float32),
        ],
    )(road_feat, idx3, lane_feat)


_NC = 2
_NS = 16
_NW = _NC * _NS
_BPW = 784


def _path_gather_sc(lane_feat, path_inverse, gt):
    mesh = plsc.VectorSubcoreMesh(core_axis_name="c", subcore_axis_name="s")

    @functools.partial(
        pl.kernel,
        mesh=mesh,
        out_type=[
            jax.ShapeDtypeStruct((N_PATH, D), jnp.float32),
            jax.ShapeDtypeStruct((N_PATH,), jnp.int32),
        ],
        scratch_types=[
            pltpu.VMEM((_BPW,), jnp.int32),
            pltpu.VMEM((_BPW, D), jnp.float32),
            pltpu.VMEM((_BPW,), jnp.int32),
            pltpu.SemaphoreType.DMA,
        ],
    )
    def k(lane_hbm, pinv_hbm, gt_hbm, plane_hbm, pgt_hbm, idx_v, rows_v,
          gt_v, sem):
        wid = lax.axis_index("s") * _NC + lax.axis_index("c")
        base = jnp.minimum(wid * _BPW, N_PATH - _BPW)
        pltpu.sync_copy(pinv_hbm.at[pl.ds(base, _BPW)], idx_v)
        gt_cp = pltpu.async_copy(gt_hbm.at[idx_v], gt_v, sem)
        rows_cp = pltpu.async_copy(lane_hbm.at[idx_v], rows_v, sem)
        gt_cp.wait()
        rows_cp.wait()
        pltpu.sync_copy(gt_v, pgt_hbm.at[pl.ds(base, _BPW)])
        pltpu.sync_copy(rows_v, plane_hbm.at[pl.ds(base, _BPW)])

    return k(lane_feat, path_inverse, gt)


_P_BLK = 5000
_P_GRID = N_PATH // _P_BLK


def _psim_body(plane_ref, mean_ref, out_ref):
    out_ref[...] = lax.dot_general(
        plane_ref[...], mean_ref[...], (((1,), (1,)), ((), ())),
        preferred_element_type=jnp.float32) * _SCALE


def _psim_tc(path_lane, road_mean):
    return pl.pallas_call(
        _psim_body,
        grid=(_P_GRID,),
        in_specs=[
            pl.BlockSpec((_P_BLK, D), lambda i: (i, 0)),
            pl.BlockSpec((NUM_ROADS, D), lambda i: (0, 0)),
        ],
        out_specs=pl.BlockSpec((_P_BLK, NUM_ROADS), lambda i: (i, 0)),
        out_shape=jax.ShapeDtypeStruct((N_PATH, NUM_ROADS), jnp.float32),
    )(path_lane, road_mean)


def kernel(lane_feat, road_feat, road_idx, path_inverse, gt):
    path_lane, path_gt = _path_gather_sc(lane_feat, path_inverse, gt)
    road_mean, sim, sim_softmax = _seg_sim_tc(road_feat, road_idx, lane_feat)
    path_sim = _psim_tc(path_lane, road_mean)
    return sim, sim_softmax, path_sim, path_gt
